# baseline (speedup 1.0000x reference)
import functools
import math

import jax
import jax.numpy as jnp
from jax import lax
from jax.experimental import pallas as pl
from jax.experimental.pallas import tpu as pltpu
from jax.experimental.pallas import tpu_sc as plsc

D_MODEL = 64
SCALE = math.sqrt(D_MODEL)

NC = 2
NS = 16
NW = NC * NS
L = 16

CHUNK = 128


def _make_gather(B: int):
    assert B % (NW * CHUNK) == 0
    b_per_w = B // NW
    n_chunks = b_per_w // CHUNK
    vecs_per_row = D_MODEL // L

    mesh = plsc.VectorSubcoreMesh(core_axis_name="c", subcore_axis_name="s")

    @functools.partial(
        pl.kernel,
        out_type=jax.ShapeDtypeStruct((B, D_MODEL), jnp.float32),
        mesh=mesh,
        compiler_params=pltpu.CompilerParams(use_tc_tiling_on_sc=False),
        scratch_types=[
            pltpu.VMEM((b_per_w,), jnp.int32),
            pltpu.VMEM((CHUNK, D_MODEL), jnp.float32),
            pltpu.SemaphoreType.DMA,
        ],
    )
    def gather_scale(x_hbm, table_hbm, out_hbm, idx_v, rows_v, gsem):
        wid = lax.axis_index("s") * NC + lax.axis_index("c")
        base = wid * b_per_w
        pltpu.sync_copy(x_hbm.at[pl.ds(base, b_per_w)], idx_v)

        @pl.loop(0, n_chunks)
        def _chunk(j):
            off = j * CHUNK
            pltpu.async_copy(
                table_hbm.at[idx_v.at[pl.ds(off, CHUNK)]], rows_v, gsem
            ).wait()

            @pl.loop(0, CHUNK)
            def _row(r):
                for v in range(vecs_per_row):
                    sl = pl.ds(v * L, L)
                    rows_v[r, sl] = rows_v[r, sl] * SCALE

            pltpu.sync_copy(rows_v, out_hbm.at[pl.ds(base + off, CHUNK)])

    return gather_scale


@jax.jit
def kernel(x, table):
    shape = x.shape
    flat = x.reshape(-1)
    out = _make_gather(flat.shape[0])(flat, table)
    return out.reshape(shape + (D_MODEL,))
